# baseline (device time: 6866 ns/iter reference)
import jax
import jax.numpy as jnp
from jax import lax
from jax.experimental import pallas as pl
from jax.experimental.pallas import tpu as pltpu

N_Y = 2


def kernel(x):
    m_per, n = x.shape

    def body(x_hbm, out_hbm, x_vmem, send_ref, in_sem, out_sem,
             send_sem, recv_sem):
        my_x = lax.axis_index("x")
        my_y = lax.axis_index("y")
        peer = (my_x, 1 - my_y)

        in_copy = pltpu.make_async_copy(x_hbm, x_vmem, in_sem)
        in_copy.start()
        barrier_sem = pltpu.get_barrier_semaphore()
        pl.semaphore_signal(
            barrier_sem, inc=1, device_id=peer,
            device_id_type=pl.DeviceIdType.MESH,
        )

        in_copy.wait()
        send_ref[...] = x_vmem[...].astype(jnp.bfloat16)

        my_off = my_y * m_per
        out_copy = pltpu.make_async_copy(
            send_ref, out_hbm.at[pl.ds(my_off, m_per)], out_sem
        )
        out_copy.start()

        pl.semaphore_wait(barrier_sem, 1)
        rdma = pltpu.make_async_remote_copy(
            src_ref=send_ref,
            dst_ref=out_hbm.at[pl.ds(my_off, m_per)],
            send_sem=send_sem,
            recv_sem=recv_sem,
            device_id=peer,
            device_id_type=pl.DeviceIdType.MESH,
        )
        rdma.start()
        out_copy.wait()
        rdma.wait()

    x = pltpu.with_memory_space_constraint(x, pltpu.MemorySpace.HBM)
    return pl.pallas_call(
        body,
        out_shape=jax.ShapeDtypeStruct((N_Y * m_per, n), jnp.bfloat16),
        in_specs=[pl.BlockSpec(memory_space=pltpu.MemorySpace.HBM)],
        out_specs=pl.BlockSpec(memory_space=pltpu.MemorySpace.HBM),
        scratch_shapes=[
            pltpu.VMEM((m_per, n), jnp.float32),
            pltpu.VMEM((m_per, n), jnp.bfloat16),
            pltpu.SemaphoreType.DMA,
            pltpu.SemaphoreType.DMA,
            pltpu.SemaphoreType.DMA,
            pltpu.SemaphoreType.DMA,
        ],
        compiler_params=pltpu.CompilerParams(collective_id=0),
    )(x)


# device time: 6644 ns/iter; 1.0334x vs baseline; 1.0334x over previous
import jax
import jax.numpy as jnp
from jax import lax
from jax.experimental import pallas as pl
from jax.experimental.pallas import tpu as pltpu

N_Y = 2


def kernel(x):
    m_per, n = x.shape

    n_chunks = 2
    m_c = m_per // n_chunks

    def body(x_hbm, out_hbm, x_vmem, send_ref, in_sems, out_sems,
             send_sems, recv_sems):
        my_x = lax.axis_index("x")
        my_y = lax.axis_index("y")
        peer = (my_x, 1 - my_y)
        my_off = my_y * m_per

        in_copies = []
        for c in range(n_chunks):
            cp = pltpu.make_async_copy(
                x_hbm.at[pl.ds(c * m_c, m_c)],
                x_vmem.at[pl.ds(c * m_c, m_c)],
                in_sems.at[c],
            )
            cp.start()
            in_copies.append(cp)
        barrier_sem = pltpu.get_barrier_semaphore()
        pl.semaphore_signal(
            barrier_sem, inc=1, device_id=peer,
            device_id_type=pl.DeviceIdType.MESH,
        )

        waits = []
        for c in range(n_chunks):
            in_copies[c].wait()
            send_ref[pl.ds(c * m_c, m_c), :] = (
                x_vmem[pl.ds(c * m_c, m_c), :].astype(jnp.bfloat16)
            )
            if c == 0:
                pl.semaphore_wait(barrier_sem, 1)
            rdma = pltpu.make_async_remote_copy(
                src_ref=send_ref.at[pl.ds(c * m_c, m_c)],
                dst_ref=out_hbm.at[pl.ds(my_off + c * m_c, m_c)],
                send_sem=send_sems.at[c],
                recv_sem=recv_sems.at[c],
                device_id=peer,
                device_id_type=pl.DeviceIdType.MESH,
            )
            rdma.start()
            out_copy = pltpu.make_async_copy(
                send_ref.at[pl.ds(c * m_c, m_c)],
                out_hbm.at[pl.ds(my_off + c * m_c, m_c)],
                out_sems.at[c],
            )
            out_copy.start()
            waits.append((rdma, out_copy))
        for rdma, out_copy in waits:
            out_copy.wait()
            rdma.wait()

    x = pltpu.with_memory_space_constraint(x, pltpu.MemorySpace.HBM)
    return pl.pallas_call(
        body,
        out_shape=jax.ShapeDtypeStruct((N_Y * m_per, n), jnp.bfloat16),
        in_specs=[pl.BlockSpec(memory_space=pltpu.MemorySpace.HBM)],
        out_specs=pl.BlockSpec(memory_space=pltpu.MemorySpace.HBM),
        scratch_shapes=[
            pltpu.VMEM((m_per, n), jnp.float32),
            pltpu.VMEM((m_per, n), jnp.bfloat16),
            pltpu.SemaphoreType.DMA((n_chunks,)),
            pltpu.SemaphoreType.DMA((n_chunks,)),
            pltpu.SemaphoreType.DMA((n_chunks,)),
            pltpu.SemaphoreType.DMA((n_chunks,)),
        ],
        compiler_params=pltpu.CompilerParams(collective_id=0),
    )(x)
